# baseline (speedup 1.0000x reference)
import functools

import jax
import jax.numpy as jnp
import numpy as np
from jax import lax
from jax.experimental import pallas as pl
from jax.experimental.pallas import tpu as pltpu
from jax.experimental.pallas import tpu_sc as plsc

_ROT0 = (13, 15, 26, 6)
_ROT1 = (17, 29, 16, 24)


def _tf2x32(k1, k2, c1, c2):
    k1 = np.asarray(k1, np.uint32)
    k2 = np.asarray(k2, np.uint32)
    x0 = np.asarray(c1, np.uint32)
    x1 = np.asarray(c2, np.uint32)
    ks2 = k1 ^ k2 ^ np.uint32(0x1BD11BDA)

    def rnds(x0, x1, rots):
        for r in rots:
            x0 = (x0 + x1).astype(np.uint32)
            x1 = ((x1 << np.uint32(r)) | (x1 >> np.uint32(32 - r))).astype(np.uint32)
            x1 = x0 ^ x1
        return x0, x1

    x0 = (x0 + k1).astype(np.uint32)
    x1 = (x1 + k2).astype(np.uint32)
    x0, x1 = rnds(x0, x1, _ROT0)
    x0 = (x0 + k2).astype(np.uint32)
    x1 = (x1 + ks2 + np.uint32(1)).astype(np.uint32)
    x0, x1 = rnds(x0, x1, _ROT1)
    x0 = (x0 + ks2).astype(np.uint32)
    x1 = (x1 + k1 + np.uint32(2)).astype(np.uint32)
    x0, x1 = rnds(x0, x1, _ROT0)
    x0 = (x0 + k1).astype(np.uint32)
    x1 = (x1 + k2 + np.uint32(3)).astype(np.uint32)
    x0, x1 = rnds(x0, x1, _ROT1)
    x0 = (x0 + k2).astype(np.uint32)
    x1 = (x1 + ks2 + np.uint32(4)).astype(np.uint32)
    x0, x1 = rnds(x0, x1, _ROT0)
    x0 = (x0 + ks2).astype(np.uint32)
    x1 = (x1 + k1 + np.uint32(5)).astype(np.uint32)
    return x0, x1


def _np_fold_in(key, d):
    a, b = _tf2x32(key[0], key[1], np.zeros(1, np.uint32),
                   np.full(1, d, np.uint32))
    return a[0], b[0]


def _np_random_bits(key, n):
    b1, b2 = _tf2x32(key[0], key[1], np.zeros(n, np.uint32),
                     np.arange(n, dtype=np.uint32))
    return b1 ^ b2


def _np_uniform(key, n):
    bits = _np_random_bits(key, n)
    fb = ((bits >> np.uint32(9)) | np.uint32(0x3F800000)).astype(np.uint32)
    return fb.view(np.float32) - np.float32(1.0)


def _np_permutation(key, n):
    x = np.arange(n)
    for _ in range(2):
        b1, b2 = _tf2x32(key[0], key[1], np.zeros(2, np.uint32),
                         np.arange(2, dtype=np.uint32))
        key, sub = (b1[0], b2[0]), (b1[1], b2[1])
        x = x[np.argsort(_np_random_bits(sub, n), kind="stable")]
    return x


B = 16384
D = 512
SUBW = 128
NSUB = 4
LW = 1000
LWP = 1024
NB = 5
NW = 32
ZROW = B
K = 32
NCH = B // (NW * K)
RPT = B // NW
OWP = NB * LWP


def _mesh():
    return plsc.VectorSubcoreMesh(core_axis_name="c", subcore_axis_name="s")


@functools.lru_cache(maxsize=None)
def _routing():
    key = (np.uint32(0), np.uint32(42))
    dec = _np_uniform(_np_fold_in(key, 0), B) < 0.5
    ps = [_np_permutation(_np_fold_in(key, i), B) for i in range(1, NSUB)]
    r = np.arange(B)

    esrc = (np.stack([r] + [np.where(dec, p, r) for p in ps])
            .astype(np.int32).reshape(NSUB, NW, NCH, K)
            .transpose(1, 0, 2, 3).copy())

    lsrc = (np.stack([np.where(dec, ZROW, r),
                      np.where(dec, r, ZROW)] +
                     [np.where(dec, p, ZROW) for p in ps])
            .astype(np.int32).reshape(NB, NW, NCH, K)
            .transpose(1, 0, 2, 3).copy())

    return esrc, lsrc


@functools.lru_cache(maxsize=None)
def _build():
    def body(embed, label_p, esrc, lsrc, out_e, out_l5,
             esrc_v, lsrc_v, eb0, eb1, lb0, lb1, esem, lsem, wsem):
        wid = lax.axis_index("s") * 2 + lax.axis_index("c")
        base = wid * RPT

        pltpu.sync_copy(esrc.at[wid], esrc_v)
        pltpu.sync_copy(lsrc.at[wid], lsrc_v)

        def egather(c, eb):
            return [pltpu.async_copy(
                embed.at[esrc_v.at[i, c], pl.ds(i * SUBW, SUBW)],
                eb.at[:, pl.ds(i * SUBW, SUBW)], esem)
                for i in range(NSUB)]

        def lgather(c, j, lb):
            return pltpu.async_copy(label_p.at[lsrc_v.at[j, c]], lb, lsem)

        def quarter(lb):
            def srow(i, carry):
                for t in range(LWP // 16):
                    lb[i, pl.ds(t * 16, 16)] = lb[i, pl.ds(t * 16, 16)] * 0.25
                return carry
            lax.fori_loop(0, K, srow, 0)

        def lwrite(c, j, lb):
            return pltpu.async_copy(
                lb, out_l5.at[pl.ds(base + c * K, K), pl.ds(j * LWP, LWP)],
                wsem)

        def chunk(c, carry):
            r0 = base + c * K
            ed = egather(c, eb0)
            g0 = lgather(c, 0, lb0)
            g1 = lgather(c, 1, lb1)
            g0.wait()
            w_prev = lwrite(c, 0, lb0)
            for j in range(1, NB):
                g1.wait()
                cur = lb1 if j % 2 else lb0
                nxt = lb0 if j % 2 else lb1
                quarter(cur)
                if j + 1 < NB:
                    w_prev.wait()
                    g1 = lgather(c, j + 1, nxt)
                w2 = lwrite(c, j, cur)
                if j + 1 >= NB:
                    w_prev.wait()
                w_prev = w2
            for d in ed:
                d.wait()
            we = pltpu.async_copy(eb0, out_e.at[pl.ds(r0, K)], esem)
            w_prev.wait()
            we.wait()
            return carry
        lax.fori_loop(0, NCH, chunk, 0)

    return pl.kernel(
        body,
        out_type=(
            jax.ShapeDtypeStruct((B, D), jnp.float32),
            jax.ShapeDtypeStruct((B, OWP), jnp.float32),
        ),
        mesh=_mesh(),
        scratch_types=[
            pltpu.VMEM((NSUB, NCH, K), jnp.int32),
            pltpu.VMEM((NB, NCH, K), jnp.int32),
            pltpu.VMEM((K, D), jnp.float32),
            pltpu.VMEM((K, D), jnp.float32),
            pltpu.VMEM((K, LWP), jnp.float32),
            pltpu.VMEM((K, LWP), jnp.float32),
            pltpu.SemaphoreType.DMA,
            pltpu.SemaphoreType.DMA,
            pltpu.SemaphoreType.DMA,
        ],
    )


def kernel(embed, onehot_label):
    esrc, lsrc = _routing()
    label_p = jnp.pad(onehot_label, ((0, 8), (0, LWP - LW)))
    out_e, out_l5 = _build()(embed, label_p, esrc, lsrc)
    out_l = out_l5.reshape(B, NB, LWP)[:, :, :LW].reshape(B, NB * LW)
    return out_e, out_l


# speedup vs baseline: 1.6453x; 1.6453x over previous
import functools

import jax
import jax.numpy as jnp
import numpy as np
from jax import lax
from jax.experimental import pallas as pl
from jax.experimental.pallas import tpu as pltpu
from jax.experimental.pallas import tpu_sc as plsc

_ROT0 = (13, 15, 26, 6)
_ROT1 = (17, 29, 16, 24)


def _tf2x32(k1, k2, c1, c2):
    k1 = np.asarray(k1, np.uint32)
    k2 = np.asarray(k2, np.uint32)
    x0 = np.asarray(c1, np.uint32)
    x1 = np.asarray(c2, np.uint32)
    ks2 = k1 ^ k2 ^ np.uint32(0x1BD11BDA)

    def rnds(x0, x1, rots):
        for r in rots:
            x0 = (x0 + x1).astype(np.uint32)
            x1 = ((x1 << np.uint32(r)) | (x1 >> np.uint32(32 - r))).astype(np.uint32)
            x1 = x0 ^ x1
        return x0, x1

    x0 = (x0 + k1).astype(np.uint32)
    x1 = (x1 + k2).astype(np.uint32)
    x0, x1 = rnds(x0, x1, _ROT0)
    x0 = (x0 + k2).astype(np.uint32)
    x1 = (x1 + ks2 + np.uint32(1)).astype(np.uint32)
    x0, x1 = rnds(x0, x1, _ROT1)
    x0 = (x0 + ks2).astype(np.uint32)
    x1 = (x1 + k1 + np.uint32(2)).astype(np.uint32)
    x0, x1 = rnds(x0, x1, _ROT0)
    x0 = (x0 + k1).astype(np.uint32)
    x1 = (x1 + k2 + np.uint32(3)).astype(np.uint32)
    x0, x1 = rnds(x0, x1, _ROT1)
    x0 = (x0 + k2).astype(np.uint32)
    x1 = (x1 + ks2 + np.uint32(4)).astype(np.uint32)
    x0, x1 = rnds(x0, x1, _ROT0)
    x0 = (x0 + ks2).astype(np.uint32)
    x1 = (x1 + k1 + np.uint32(5)).astype(np.uint32)
    return x0, x1


def _np_fold_in(key, d):
    a, b = _tf2x32(key[0], key[1], np.zeros(1, np.uint32),
                   np.full(1, d, np.uint32))
    return a[0], b[0]


def _np_random_bits(key, n):
    b1, b2 = _tf2x32(key[0], key[1], np.zeros(n, np.uint32),
                     np.arange(n, dtype=np.uint32))
    return b1 ^ b2


def _np_uniform(key, n):
    bits = _np_random_bits(key, n)
    fb = ((bits >> np.uint32(9)) | np.uint32(0x3F800000)).astype(np.uint32)
    return fb.view(np.float32) - np.float32(1.0)


def _np_permutation(key, n):
    x = np.arange(n)
    for _ in range(2):
        b1, b2 = _tf2x32(key[0], key[1], np.zeros(2, np.uint32),
                         np.arange(2, dtype=np.uint32))
        key, sub = (b1[0], b2[0]), (b1[1], b2[1])
        x = x[np.argsort(_np_random_bits(sub, n), kind="stable")]
    return x


B = 16384
D = 512
SUBW = 128
NSUB = 4
LW = 1000
NB = 5
NW = 32
KE = 64
ECH = B // (NW * KE)
KL = 32
RPT = B // NW


def _mesh():
    return plsc.VectorSubcoreMesh(core_axis_name="c", subcore_axis_name="s")


@functools.lru_cache(maxsize=None)
def _routing():
    key = (np.uint32(0), np.uint32(42))
    dec = _np_uniform(_np_fold_in(key, 0), B) < 0.5
    ps = [_np_permutation(_np_fold_in(key, i), B) for i in range(1, NSUB)]
    r = np.arange(B)

    esrc = (np.stack([r] + [np.where(dec, p, r) for p in ps])
            .astype(np.int32).reshape(NSUB, NW, ECH, KE)
            .transpose(1, 0, 2, 3).copy())

    r0s = r[~dec]
    r1s = r[dec]
    copy_out, copy_src = NB * r0s, r0s
    quar_out = np.concatenate([NB * r1s + j for j in range(1, NB)])
    quar_src = np.concatenate([r1s] + [p[r1s] for p in ps])
    zero_out = np.concatenate(
        [NB * r1s, (NB * r0s[:, None] + np.arange(1, NB)[None, :]).reshape(-1)])

    def pad(a):
        m = 2 * NW * KL
        n = -len(a) % m
        return (np.concatenate([a, np.repeat(a[-1:], n)])
                .astype(np.int32).reshape(NW, -1, KL))

    half = B * NB // 2
    halves = []
    for lo in (0, half):
        hi = lo + half
        mc = (copy_out >= lo) & (copy_out < hi)
        mq = (quar_out >= lo) & (quar_out < hi)
        mz = (zero_out >= lo) & (zero_out < hi)
        halves.append((pad(copy_src[mc]), pad(copy_out[mc] - lo),
                       pad(quar_src[mq]), pad(quar_out[mq] - lo),
                       pad(zero_out[mz] - lo)))
    return esrc, halves


@functools.lru_cache(maxsize=None)
def _build_embed():
    def body(embed, esrc, out_e, esrc_v, eb0, eb1, gsem, wsem):
        wid = lax.axis_index("s") * 2 + lax.axis_index("c")
        base = wid * RPT
        pltpu.sync_copy(esrc.at[wid], esrc_v)

        def gather(c, eb):
            return [pltpu.async_copy(
                embed.at[esrc_v.at[i, c], pl.ds(i * SUBW, SUBW)],
                eb.at[:, pl.ds(i * SUBW, SUBW)], gsem)
                for i in range(NSUB)]

        def pair(p, carry):
            c0 = 2 * p
            g0 = gather(c0, eb0)
            for d in g0:
                d.wait()
            w0 = pltpu.async_copy(eb0, out_e.at[pl.ds(base + c0 * KE, KE)], wsem)
            g1 = gather(c0 + 1, eb1)
            for d in g1:
                d.wait()
            w0.wait()
            w1 = pltpu.async_copy(eb1, out_e.at[pl.ds(base + (c0 + 1) * KE, KE)], wsem)
            w1.wait()
            return carry
        lax.fori_loop(0, ECH // 2, pair, 0)

    return pl.kernel(
        body,
        out_type=jax.ShapeDtypeStruct((B, D), jnp.float32),
        mesh=_mesh(),
        scratch_types=[
            pltpu.VMEM((NSUB, ECH, KE), jnp.int32),
            pltpu.VMEM((KE, D), jnp.float32),
            pltpu.VMEM((KE, D), jnp.float32),
            pltpu.SemaphoreType.DMA,
            pltpu.SemaphoreType.DMA,
        ],
    )


@functools.lru_cache(maxsize=None)
def _build_label(ncc, nqc, nzc):
    def body(label, cs, co, qs, qo, zo, out_l,
             cs_v, co_v, qs_v, qo_v, zo_v, zbuf, ba, bb, gsem, ssem, zsem):
        wid = lax.axis_index("s") * 2 + lax.axis_index("c")

        pltpu.sync_copy(cs.at[wid], cs_v)
        pltpu.sync_copy(co.at[wid], co_v)
        pltpu.sync_copy(qs.at[wid], qs_v)
        pltpu.sync_copy(qo.at[wid], qo_v)
        pltpu.sync_copy(zo.at[wid], zo_v)

        z16 = jnp.zeros((16,), jnp.float32)

        def zrow(i, carry):
            for t in range(LW // 16):
                zbuf[i, pl.ds(t * 16, 16)] = z16
            zbuf[i, pl.ds(LW - 16, 16)] = z16
            return carry
        lax.fori_loop(0, KL, zrow, 0)

        zdescs = [pltpu.async_copy(zbuf, out_l.at[zo_v.at[z]], zsem)
                  for z in range(nzc)]

        def cpair(p, carry):
            c0 = 2 * p
            pltpu.async_copy(label.at[cs_v.at[c0]], ba, gsem).wait()
            sa = pltpu.async_copy(ba, out_l.at[co_v.at[c0]], ssem)
            pltpu.async_copy(label.at[cs_v.at[c0 + 1]], bb, gsem).wait()
            sa.wait()
            pltpu.async_copy(bb, out_l.at[co_v.at[c0 + 1]], ssem).wait()
            return carry
        lax.fori_loop(0, ncc // 2, cpair, 0)

        tailsel = jnp.arange(16) >= 8

        def scale(buf):
            def srow(i, carry):
                for t in range(LW // 16):
                    buf[i, pl.ds(t * 16, 16)] = buf[i, pl.ds(t * 16, 16)] * 0.25
                v = buf[i, pl.ds(LW - 16, 16)]
                buf[i, pl.ds(LW - 16, 16)] = jnp.where(tailsel, v * 0.25, v)
                return carry
            lax.fori_loop(0, KL, srow, 0)

        def qpair(p, carry):
            c0 = 2 * p
            pltpu.async_copy(label.at[qs_v.at[c0]], ba, gsem).wait()
            scale(ba)
            sa = pltpu.async_copy(ba, out_l.at[qo_v.at[c0]], ssem)
            pltpu.async_copy(label.at[qs_v.at[c0 + 1]], bb, gsem).wait()
            scale(bb)
            sa.wait()
            pltpu.async_copy(bb, out_l.at[qo_v.at[c0 + 1]], ssem).wait()
            return carry
        lax.fori_loop(0, nqc // 2, qpair, 0)

        for d in zdescs:
            d.wait()

    return pl.kernel(
        body,
        compiler_params=pltpu.CompilerParams(use_tc_tiling_on_sc=False),
        out_type=jax.ShapeDtypeStruct((B * NB // 2, LW), jnp.float32),
        mesh=_mesh(),
        scratch_types=[
            pltpu.VMEM((ncc, KL), jnp.int32),
            pltpu.VMEM((ncc, KL), jnp.int32),
            pltpu.VMEM((nqc, KL), jnp.int32),
            pltpu.VMEM((nqc, KL), jnp.int32),
            pltpu.VMEM((nzc, KL), jnp.int32),
            pltpu.VMEM((KL, LW), jnp.float32),
            pltpu.VMEM((KL, LW), jnp.float32),
            pltpu.VMEM((KL, LW), jnp.float32),
            pltpu.SemaphoreType.DMA,
            pltpu.SemaphoreType.DMA,
            pltpu.SemaphoreType.DMA,
        ],
    )


def kernel(embed, onehot_label):
    esrc, halves = _routing()
    out_e = _build_embed()(embed, esrc)
    hs = [_build_label(cs.shape[1], qs.shape[1], zo.shape[1])(
        onehot_label, cs, co, qs, qo, zo)
        for cs, co, qs, qo, zo in halves]
    out_l = jnp.concatenate(hs, axis=0).reshape(B, NB * LW)
    return out_e, out_l


# speedup vs baseline: 2.8237x; 1.7162x over previous
import functools

import jax
import jax.numpy as jnp
import numpy as np
from jax import lax
from jax.experimental import pallas as pl
from jax.experimental.pallas import tpu as pltpu
from jax.experimental.pallas import tpu_sc as plsc

_ROT0 = (13, 15, 26, 6)
_ROT1 = (17, 29, 16, 24)


def _tf2x32(k1, k2, c1, c2):
    k1 = np.asarray(k1, np.uint32)
    k2 = np.asarray(k2, np.uint32)
    x0 = np.asarray(c1, np.uint32)
    x1 = np.asarray(c2, np.uint32)
    ks2 = k1 ^ k2 ^ np.uint32(0x1BD11BDA)

    def rnds(x0, x1, rots):
        for r in rots:
            x0 = (x0 + x1).astype(np.uint32)
            x1 = ((x1 << np.uint32(r)) | (x1 >> np.uint32(32 - r))).astype(np.uint32)
            x1 = x0 ^ x1
        return x0, x1

    x0 = (x0 + k1).astype(np.uint32)
    x1 = (x1 + k2).astype(np.uint32)
    x0, x1 = rnds(x0, x1, _ROT0)
    x0 = (x0 + k2).astype(np.uint32)
    x1 = (x1 + ks2 + np.uint32(1)).astype(np.uint32)
    x0, x1 = rnds(x0, x1, _ROT1)
    x0 = (x0 + ks2).astype(np.uint32)
    x1 = (x1 + k1 + np.uint32(2)).astype(np.uint32)
    x0, x1 = rnds(x0, x1, _ROT0)
    x0 = (x0 + k1).astype(np.uint32)
    x1 = (x1 + k2 + np.uint32(3)).astype(np.uint32)
    x0, x1 = rnds(x0, x1, _ROT1)
    x0 = (x0 + k2).astype(np.uint32)
    x1 = (x1 + ks2 + np.uint32(4)).astype(np.uint32)
    x0, x1 = rnds(x0, x1, _ROT0)
    x0 = (x0 + ks2).astype(np.uint32)
    x1 = (x1 + k1 + np.uint32(5)).astype(np.uint32)
    return x0, x1


def _np_fold_in(key, d):
    a, b = _tf2x32(key[0], key[1], np.zeros(1, np.uint32),
                   np.full(1, d, np.uint32))
    return a[0], b[0]


def _np_random_bits(key, n):
    b1, b2 = _tf2x32(key[0], key[1], np.zeros(n, np.uint32),
                     np.arange(n, dtype=np.uint32))
    return b1 ^ b2


def _np_uniform(key, n):
    bits = _np_random_bits(key, n)
    fb = ((bits >> np.uint32(9)) | np.uint32(0x3F800000)).astype(np.uint32)
    return fb.view(np.float32) - np.float32(1.0)


def _np_permutation(key, n):
    x = np.arange(n)
    for _ in range(2):
        b1, b2 = _tf2x32(key[0], key[1], np.zeros(2, np.uint32),
                         np.arange(2, dtype=np.uint32))
        key, sub = (b1[0], b2[0]), (b1[1], b2[1])
        x = x[np.argsort(_np_random_bits(sub, n), kind="stable")]
    return x


B = 16384
D = 512
SUBW = 128
NSUB = 4
LW = 1000
NB = 5
NW = 32
KE = 64
ECH = B // (NW * KE)
KL = 32
RPT = B // NW


def _mesh():
    return plsc.VectorSubcoreMesh(core_axis_name="c", subcore_axis_name="s")


@functools.lru_cache(maxsize=None)
def _routing():
    key = (np.uint32(0), np.uint32(42))
    dec = _np_uniform(_np_fold_in(key, 0), B) < 0.5
    ps = [_np_permutation(_np_fold_in(key, i), B) for i in range(1, NSUB)]
    r = np.arange(B)

    esrc = (np.stack([r] + [np.where(dec, p, r) for p in ps])
            .astype(np.int32).reshape(NSUB, NW, ECH, KE)
            .transpose(1, 0, 2, 3).copy())

    r0s = r[~dec]
    r1s = r[dec]
    copy_out, copy_src = NB * r0s, r0s
    quar_out = np.concatenate([NB * r1s + j for j in range(1, NB)])
    quar_src = np.concatenate([r1s] + [p[r1s] for p in ps])
    zero_out = np.concatenate(
        [NB * r1s, (NB * r0s[:, None] + np.arange(1, NB)[None, :]).reshape(-1)])

    def pad(a):
        m = 2 * NW * KL
        n = -len(a) % m
        return (np.concatenate([a, np.repeat(a[-1:], n)])
                .astype(np.int32).reshape(NW, -1, KL))

    return (esrc, pad(copy_src), pad(copy_out),
            pad(quar_src), pad(quar_out), pad(zero_out))


@functools.lru_cache(maxsize=None)
def _build_embed():
    def body(embed, esrc, out_e, esrc_v, eb0, eb1, gsem, wsem):
        wid = lax.axis_index("s") * 2 + lax.axis_index("c")
        base = wid * RPT
        pltpu.sync_copy(esrc.at[wid], esrc_v)

        def gather(c, eb):
            return [pltpu.async_copy(
                embed.at[esrc_v.at[i, c], pl.ds(i * SUBW, SUBW)],
                eb.at[:, pl.ds(i * SUBW, SUBW)], gsem)
                for i in range(NSUB)]

        def pair(p, carry):
            c0 = 2 * p
            g0 = gather(c0, eb0)
            for d in g0:
                d.wait()
            w0 = pltpu.async_copy(eb0, out_e.at[pl.ds(base + c0 * KE, KE)], wsem)
            g1 = gather(c0 + 1, eb1)
            for d in g1:
                d.wait()
            w0.wait()
            w1 = pltpu.async_copy(eb1, out_e.at[pl.ds(base + (c0 + 1) * KE, KE)], wsem)
            w1.wait()
            return carry
        lax.fori_loop(0, ECH // 2, pair, 0)

    return pl.kernel(
        body,
        out_type=jax.ShapeDtypeStruct((B, D), jnp.float32),
        mesh=_mesh(),
        scratch_types=[
            pltpu.VMEM((NSUB, ECH, KE), jnp.int32),
            pltpu.VMEM((KE, D), jnp.float32),
            pltpu.VMEM((KE, D), jnp.float32),
            pltpu.SemaphoreType.DMA,
            pltpu.SemaphoreType.DMA,
        ],
    )


@functools.lru_cache(maxsize=None)
def _build_label(ncc, nqc, nzc):
    def body(label, cs, co, qs, qo, zo, out_l,
             cs_v, co_v, qs_v, qo_v, zo_v, zbuf, ba, bb, gsem, ssem, zsem):
        wid = lax.axis_index("s") * 2 + lax.axis_index("c")

        pltpu.sync_copy(cs.at[wid], cs_v)
        pltpu.sync_copy(co.at[wid], co_v)
        pltpu.sync_copy(qs.at[wid], qs_v)
        pltpu.sync_copy(qo.at[wid], qo_v)
        pltpu.sync_copy(zo.at[wid], zo_v)

        z16 = jnp.zeros((16,), jnp.float32)

        def zrow(i, carry):
            for t in range(LW // 16):
                zbuf[i, pl.ds(t * 16, 16)] = z16
            zbuf[i, pl.ds(LW - 16, 16)] = z16
            return carry
        lax.fori_loop(0, KL, zrow, 0)

        zdescs = [pltpu.async_copy(zbuf, out_l.at[zo_v.at[z]], zsem)
                  for z in range(nzc)]

        def cpair(p, carry):
            c0 = 2 * p
            ga = pltpu.async_copy(label.at[cs_v.at[c0]], ba, gsem)
            gb = pltpu.async_copy(label.at[cs_v.at[c0 + 1]], bb, gsem)
            ga.wait()
            sa = pltpu.async_copy(ba, out_l.at[co_v.at[c0]], ssem)
            gb.wait()
            sb = pltpu.async_copy(bb, out_l.at[co_v.at[c0 + 1]], ssem)
            sa.wait()
            sb.wait()
            return carry
        lax.fori_loop(0, ncc // 2, cpair, 0)

        tailsel = jnp.arange(16) >= 8

        def scale(buf):
            def srow(i, carry):
                for t in range(LW // 16):
                    buf[i, pl.ds(t * 16, 16)] = buf[i, pl.ds(t * 16, 16)] * 0.25
                v = buf[i, pl.ds(LW - 16, 16)]
                buf[i, pl.ds(LW - 16, 16)] = jnp.where(tailsel, v * 0.25, v)
                return carry
            lax.fori_loop(0, KL, srow, 0)

        def qpair(p, carry):
            c0 = 2 * p
            ga = pltpu.async_copy(label.at[qs_v.at[c0]], ba, gsem)
            gb = pltpu.async_copy(label.at[qs_v.at[c0 + 1]], bb, gsem)
            ga.wait()
            scale(ba)
            sa = pltpu.async_copy(ba, out_l.at[qo_v.at[c0]], ssem)
            gb.wait()
            scale(bb)
            sa.wait()
            sb = pltpu.async_copy(bb, out_l.at[qo_v.at[c0 + 1]], ssem)
            sb.wait()
            return carry
        lax.fori_loop(0, nqc // 2, qpair, 0)

        for d in zdescs:
            d.wait()

    return pl.kernel(
        body,
        compiler_params=pltpu.CompilerParams(use_tc_tiling_on_sc=False),
        out_type=jax.ShapeDtypeStruct((B * NB, LW), jnp.float32),
        mesh=_mesh(),
        scratch_types=[
            pltpu.VMEM((ncc, KL), jnp.int32),
            pltpu.VMEM((ncc, KL), jnp.int32),
            pltpu.VMEM((nqc, KL), jnp.int32),
            pltpu.VMEM((nqc, KL), jnp.int32),
            pltpu.VMEM((nzc, KL), jnp.int32),
            pltpu.VMEM((KL, LW), jnp.float32),
            pltpu.VMEM((KL, LW), jnp.float32),
            pltpu.VMEM((KL, LW), jnp.float32),
            pltpu.SemaphoreType.DMA,
            pltpu.SemaphoreType.DMA,
            pltpu.SemaphoreType.DMA,
        ],
    )


def kernel(embed, onehot_label):
    esrc, cs, co, qs, qo, zo = _routing()
    out_e = _build_embed()(embed, esrc)
    out_l = _build_label(cs.shape[1], qs.shape[1], zo.shape[1])(
        onehot_label, cs, co, qs, qo, zo)
    return out_e, out_l.reshape(B, NB * LW)


# speedup vs baseline: 2.8281x; 1.0016x over previous
import functools

import jax
import jax.numpy as jnp
import numpy as np
from jax import lax
from jax.experimental import pallas as pl
from jax.experimental.pallas import tpu as pltpu
from jax.experimental.pallas import tpu_sc as plsc

_ROT0 = (13, 15, 26, 6)
_ROT1 = (17, 29, 16, 24)


def _tf2x32(k1, k2, c1, c2):
    k1 = np.asarray(k1, np.uint32)
    k2 = np.asarray(k2, np.uint32)
    x0 = np.asarray(c1, np.uint32)
    x1 = np.asarray(c2, np.uint32)
    ks2 = k1 ^ k2 ^ np.uint32(0x1BD11BDA)

    def rnds(x0, x1, rots):
        for r in rots:
            x0 = (x0 + x1).astype(np.uint32)
            x1 = ((x1 << np.uint32(r)) | (x1 >> np.uint32(32 - r))).astype(np.uint32)
            x1 = x0 ^ x1
        return x0, x1

    x0 = (x0 + k1).astype(np.uint32)
    x1 = (x1 + k2).astype(np.uint32)
    x0, x1 = rnds(x0, x1, _ROT0)
    x0 = (x0 + k2).astype(np.uint32)
    x1 = (x1 + ks2 + np.uint32(1)).astype(np.uint32)
    x0, x1 = rnds(x0, x1, _ROT1)
    x0 = (x0 + ks2).astype(np.uint32)
    x1 = (x1 + k1 + np.uint32(2)).astype(np.uint32)
    x0, x1 = rnds(x0, x1, _ROT0)
    x0 = (x0 + k1).astype(np.uint32)
    x1 = (x1 + k2 + np.uint32(3)).astype(np.uint32)
    x0, x1 = rnds(x0, x1, _ROT1)
    x0 = (x0 + k2).astype(np.uint32)
    x1 = (x1 + ks2 + np.uint32(4)).astype(np.uint32)
    x0, x1 = rnds(x0, x1, _ROT0)
    x0 = (x0 + ks2).astype(np.uint32)
    x1 = (x1 + k1 + np.uint32(5)).astype(np.uint32)
    return x0, x1


def _np_fold_in(key, d):
    a, b = _tf2x32(key[0], key[1], np.zeros(1, np.uint32),
                   np.full(1, d, np.uint32))
    return a[0], b[0]


def _np_random_bits(key, n):
    b1, b2 = _tf2x32(key[0], key[1], np.zeros(n, np.uint32),
                     np.arange(n, dtype=np.uint32))
    return b1 ^ b2


def _np_uniform(key, n):
    bits = _np_random_bits(key, n)
    fb = ((bits >> np.uint32(9)) | np.uint32(0x3F800000)).astype(np.uint32)
    return fb.view(np.float32) - np.float32(1.0)


def _np_permutation(key, n):
    x = np.arange(n)
    for _ in range(2):
        b1, b2 = _tf2x32(key[0], key[1], np.zeros(2, np.uint32),
                         np.arange(2, dtype=np.uint32))
        key, sub = (b1[0], b2[0]), (b1[1], b2[1])
        x = x[np.argsort(_np_random_bits(sub, n), kind="stable")]
    return x


B = 16384
D = 512
SUBW = 128
NSUB = 4
LW = 1000
NB = 5
NW = 32
KE = 64
ECH = B // (NW * KE)
KL = 32
RPT = B // NW


def _mesh():
    return plsc.VectorSubcoreMesh(core_axis_name="c", subcore_axis_name="s")


@functools.lru_cache(maxsize=None)
def _routing():
    key = (np.uint32(0), np.uint32(42))
    dec = _np_uniform(_np_fold_in(key, 0), B) < 0.5
    ps = [_np_permutation(_np_fold_in(key, i), B) for i in range(1, NSUB)]
    r = np.arange(B)

    esrc = (np.stack([r] + [np.where(dec, p, r) for p in ps])
            .astype(np.int32).reshape(NSUB, NW, ECH, KE)
            .transpose(1, 0, 2, 3).copy())

    r0s = r[~dec]
    r1s = r[dec]
    copy_out, copy_src = NB * r0s, r0s
    quar_out = np.concatenate([NB * r1s + j for j in range(1, NB)])
    quar_src = np.concatenate([r1s] + [p[r1s] for p in ps])
    zero_out = np.concatenate(
        [NB * r1s, (NB * r0s[:, None] + np.arange(1, NB)[None, :]).reshape(-1)])

    def pad(a):
        m = 2 * NW * KL
        n = -len(a) % m
        return (np.concatenate([a, np.repeat(a[-1:], n)])
                .astype(np.int32).reshape(NW, -1, KL))

    return (esrc, pad(copy_src), pad(copy_out),
            pad(quar_src), pad(quar_out), pad(zero_out))


@functools.lru_cache(maxsize=None)
def _build_embed():
    def body(embed, esrc, out_e, esrc_v, eb0, eb1, gsem, wsem):
        wid = lax.axis_index("s") * 2 + lax.axis_index("c")
        base = wid * RPT
        pltpu.sync_copy(esrc.at[wid], esrc_v)

        def gather(c, eb):
            return [pltpu.async_copy(
                embed.at[esrc_v.at[i, c], pl.ds(i * SUBW, SUBW)],
                eb.at[:, pl.ds(i * SUBW, SUBW)], gsem)
                for i in range(NSUB)]

        def pair(p, carry):
            c0 = 2 * p
            g0 = gather(c0, eb0)
            for d in g0:
                d.wait()
            w0 = pltpu.async_copy(eb0, out_e.at[pl.ds(base + c0 * KE, KE)], wsem)
            g1 = gather(c0 + 1, eb1)
            for d in g1:
                d.wait()
            w0.wait()
            w1 = pltpu.async_copy(eb1, out_e.at[pl.ds(base + (c0 + 1) * KE, KE)], wsem)
            w1.wait()
            return carry
        lax.fori_loop(0, ECH // 2, pair, 0)

    return pl.kernel(
        body,
        out_type=jax.ShapeDtypeStruct((B, D), jnp.float32),
        mesh=_mesh(),
        scratch_types=[
            pltpu.VMEM((NSUB, ECH, KE), jnp.int32),
            pltpu.VMEM((KE, D), jnp.float32),
            pltpu.VMEM((KE, D), jnp.float32),
            pltpu.SemaphoreType.DMA,
            pltpu.SemaphoreType.DMA,
        ],
    )


@functools.lru_cache(maxsize=None)
def _build_label(ncc, nqc, nzc):
    def body(label, cs, co, qs, qo, zo, out_l,
             cs_v, co_v, qs_v, qo_v, zo_v, zbuf, ba, bb, gsem, ssem, zsem):
        wid = lax.axis_index("s") * 2 + lax.axis_index("c")

        pltpu.sync_copy(cs.at[wid], cs_v)
        pltpu.sync_copy(co.at[wid], co_v)
        pltpu.sync_copy(qs.at[wid], qs_v)
        pltpu.sync_copy(qo.at[wid], qo_v)
        pltpu.sync_copy(zo.at[wid], zo_v)

        z16 = jnp.zeros((16,), jnp.float32)

        def zrow(i, carry):
            for t in range(LW // 16):
                zbuf[i, pl.ds(t * 16, 16)] = z16
            zbuf[i, pl.ds(LW - 16, 16)] = z16
            return carry
        lax.fori_loop(0, KL, zrow, 0)

        zdescs = [pltpu.async_copy(zbuf, out_l.at[zo_v.at[z]], zsem)
                  for z in range(nzc)]

        def drain2(idx_v):
            pltpu.make_async_copy(ba, out_l.at[idx_v.at[0]], ssem).wait()
            pltpu.make_async_copy(bb, out_l.at[idx_v.at[0]], ssem).wait()

        def cwork(c0):
            ga = pltpu.async_copy(label.at[cs_v.at[c0]], ba, gsem)
            gb = pltpu.async_copy(label.at[cs_v.at[c0 + 1]], bb, gsem)
            ga.wait()
            pltpu.async_copy(ba, out_l.at[co_v.at[c0]], ssem)
            gb.wait()
            pltpu.async_copy(bb, out_l.at[co_v.at[c0 + 1]], ssem)

        cwork(0)

        def cpair(p, carry):
            drain2(co_v)
            cwork(2 * p)
            return carry
        lax.fori_loop(1, ncc // 2, cpair, 0)
        drain2(co_v)

        tailsel = jnp.arange(16) >= 8

        def scale(buf):
            def srow(i, carry):
                for t in range(LW // 16):
                    buf[i, pl.ds(t * 16, 16)] = buf[i, pl.ds(t * 16, 16)] * 0.25
                v = buf[i, pl.ds(LW - 16, 16)]
                buf[i, pl.ds(LW - 16, 16)] = jnp.where(tailsel, v * 0.25, v)
                return carry
            lax.fori_loop(0, KL, srow, 0)

        def qwork(c0):
            ga = pltpu.async_copy(label.at[qs_v.at[c0]], ba, gsem)
            gb = pltpu.async_copy(label.at[qs_v.at[c0 + 1]], bb, gsem)
            ga.wait()
            scale(ba)
            pltpu.async_copy(ba, out_l.at[qo_v.at[c0]], ssem)
            gb.wait()
            scale(bb)
            pltpu.async_copy(bb, out_l.at[qo_v.at[c0 + 1]], ssem)

        qwork(0)

        def qpair(p, carry):
            drain2(qo_v)
            qwork(2 * p)
            return carry
        lax.fori_loop(1, nqc // 2, qpair, 0)
        drain2(qo_v)

        for d in zdescs:
            d.wait()

    return pl.kernel(
        body,
        compiler_params=pltpu.CompilerParams(use_tc_tiling_on_sc=False),
        out_type=jax.ShapeDtypeStruct((B * NB, LW), jnp.float32),
        mesh=_mesh(),
        scratch_types=[
            pltpu.VMEM((ncc, KL), jnp.int32),
            pltpu.VMEM((ncc, KL), jnp.int32),
            pltpu.VMEM((nqc, KL), jnp.int32),
            pltpu.VMEM((nqc, KL), jnp.int32),
            pltpu.VMEM((nzc, KL), jnp.int32),
            pltpu.VMEM((KL, LW), jnp.float32),
            pltpu.VMEM((KL, LW), jnp.float32),
            pltpu.VMEM((KL, LW), jnp.float32),
            pltpu.SemaphoreType.DMA,
            pltpu.SemaphoreType.DMA,
            pltpu.SemaphoreType.DMA,
        ],
    )


def kernel(embed, onehot_label):
    esrc, cs, co, qs, qo, zo = _routing()
    out_e = _build_embed()(embed, esrc)
    out_l = _build_label(cs.shape[1], qs.shape[1], zo.shape[1])(
        onehot_label, cs, co, qs, qo, zo)
    return out_e, out_l.reshape(B, NB * LW)
